# baseline (speedup 1.0000x reference)
import jax
import jax.numpy as jnp
from jax.experimental import pallas as pl

RATIO = 0.8

ROW_BLK = 1000


def _proj_kernel(x_ref, w_ref, o_ref):
    o_ref[...] = jnp.dot(x_ref[...], w_ref[...],
                         preferred_element_type=jnp.float32)


def _proj(x, w):
    n, d = x.shape
    dout = w.shape[1]
    grid = n // ROW_BLK
    return pl.pallas_call(
        _proj_kernel,
        grid=(grid,),
        in_specs=[
            pl.BlockSpec((ROW_BLK, d), lambda i: (i, 0)),
            pl.BlockSpec((d, dout), lambda i: (0, 0)),
        ],
        out_specs=pl.BlockSpec((ROW_BLK, dout), lambda i: (i, 0)),
        out_shape=jax.ShapeDtypeStruct((n, dout), jnp.float32),
    )(x, w)


def _head_kernel(g_ref, le_ref, wm_ref, bm_ref, wlv_ref, blv_ref,
                 wd1_ref, bd1_ref, wd2_ref, bd2_ref, wo_ref, bo_ref,
                 out_ref, mu_ref, lv_ref):
    g = g_ref[...]
    le = jnp.maximum(le_ref[...], 0.0)
    mu = jnp.maximum(
        jnp.dot(g, wm_ref[...], preferred_element_type=jnp.float32)
        + bm_ref[...], 0.0)
    lv = jnp.maximum(
        jnp.dot(g, wlv_ref[...], preferred_element_type=jnp.float32)
        + blv_ref[...], 0.0)
    zc = jnp.concatenate([mu, le], axis=1)
    h1 = jnp.maximum(
        jnp.dot(zc, wd1_ref[...], preferred_element_type=jnp.float32)
        + bd1_ref[...], 0.0)
    h2 = jnp.maximum(
        jnp.dot(h1, wd2_ref[...], preferred_element_type=jnp.float32)
        + bd2_ref[...], 0.0) + le
    out = jnp.maximum(
        jnp.dot(h2, wo_ref[...], preferred_element_type=jnp.float32)
        + bo_ref[...], 0.0)
    out_ref[...] = out
    mu_ref[...] = mu
    lv_ref[...] = lv


def _head(g, le_raw, Wm, bm, Wlv, blv, Wd1, bd1, Wd2, bd2, Wo, bo):
    nb = g.shape[0]
    lat = Wm.shape[1]
    smile = Wo.shape[1]
    full = lambda s: pl.BlockSpec(s, lambda: (0, 0))
    args = [g, le_raw,
            Wm, bm.reshape(1, -1), Wlv, blv.reshape(1, -1),
            Wd1, bd1.reshape(1, -1), Wd2, bd2.reshape(1, -1),
            Wo, bo.reshape(1, -1)]
    return pl.pallas_call(
        _head_kernel,
        in_specs=[full(a.shape) for a in args],
        out_specs=[full((nb, smile)), full((nb, lat)), full((nb, lat))],
        out_shape=[
            jax.ShapeDtypeStruct((nb, smile), jnp.float32),
            jax.ShapeDtypeStruct((nb, lat), jnp.float32),
            jax.ShapeDtypeStruct((nb, lat), jnp.float32),
        ],
    )(*args)


def _gcn_conv(x, ei, W, b):
    n = x.shape[0]
    loop = jnp.arange(n, dtype=ei.dtype)
    src = jnp.concatenate([ei[0], loop])
    dst = jnp.concatenate([ei[1], loop])
    deg = jnp.zeros((n,), jnp.float32).at[dst].add(1.0)
    dis = jax.lax.rsqrt(deg)
    norm = dis[src] * dis[dst]
    h = _proj(x, W)
    out = jnp.zeros((n, W.shape[1]), h.dtype).at[dst].add(h[src] * norm[:, None])
    return out + b


def _topk_pool(x, ei, batch, p, num_graphs, mask):
    score = jnp.tanh((x @ p) / jnp.linalg.norm(p))
    n = x.shape[0]
    b_ext = jnp.where(mask, batch, num_graphs)
    counts = jnp.bincount(b_ext, length=num_graphs)
    k = jnp.ceil(RATIO * counts.astype(jnp.float32)).astype(jnp.int32)
    order = jnp.lexsort((-score, b_ext))
    starts = jnp.concatenate([jnp.zeros((1,), counts.dtype),
                              jnp.cumsum(counts)[:-1]])
    sb = b_ext[order]
    rank = jnp.arange(n) - starts[sb]
    keep = mask[order] & (rank < k[sb])
    new_x = jnp.where(keep[:, None], x[order] * score[order][:, None], 0.0)
    new_batch = jnp.where(keep, batch[order], num_graphs)
    node_mask = jnp.zeros((n,), bool).at[order].set(keep)
    new_id = jnp.zeros((n,), jnp.int32).at[order].set(
        jnp.arange(n, dtype=jnp.int32))
    in_range = (ei[0] >= 0) & (ei[0] < n) & (ei[1] >= 0) & (ei[1] < n)
    emask = in_range & node_mask[ei[0]] & node_mask[ei[1]]
    new_ei = jnp.where(emask, new_id[ei], jnp.asarray(n, new_id.dtype))
    return new_x, new_ei, new_batch, keep


def kernel(x, edge_index, batch, len, W1, b1, p1, W2, b2, p2, Wm, bm,
           Wlv, blv, Emb, Wd1, bd1, Wd2, bd2, Wo, bo):
    len_idx = len
    num_graphs = len_idx.shape[0]
    mask0 = jnp.ones((x.shape[0],), bool)
    x1 = jax.nn.relu(_gcn_conv(x, edge_index, W1, b1))
    xp1, ei1, batch1, m1 = _topk_pool(x1, edge_index, batch, p1,
                                      num_graphs, mask0)
    x2 = jax.nn.relu(_gcn_conv(xp1, ei1, W2, b2))
    xp2, ei2, batch2, m2 = _topk_pool(x2, ei1, batch1, p2, num_graphs, m1)
    gmax = jax.ops.segment_max(xp2, batch2, num_segments=num_graphs)
    cnt = jnp.bincount(batch2, length=num_graphs).astype(jnp.float32)
    gmean = (jax.ops.segment_sum(xp2, batch2, num_segments=num_graphs)
             / jnp.maximum(cnt, 1.0)[:, None])
    g = jnp.concatenate([gmax, gmean], axis=1)
    le_raw = Emb[len_idx]
    out, mu, logvar = _head(g, le_raw, Wm, bm, Wlv, blv,
                            Wd1, bd1, Wd2, bd2, Wo, bo)
    return (out, mu, logvar)
